# Initial kernel scaffold; baseline (speedup 1.0000x reference)
import jax
import jax.numpy as jnp
from jax.experimental import pallas as pl


def kernel(s, nrow_gt, W1):
    raise NotImplementedError("write your pallas kernel here")


# speedup vs baseline: 1.1224x; 1.1224x over previous
import functools

import jax
import jax.numpy as jnp
from jax.experimental import pallas as pl
from jax.experimental.pallas import tpu as pltpu

ALPHA = 200.0
BR = 256


def _body(nrow_ref, s_ref, w_ref, o_ref):
    b = pl.program_id(0)
    r = pl.program_id(1)
    nrow = nrow_ref[b]
    base = r * BR
    row_ids = base + jax.lax.broadcasted_iota(jnp.int32, (BR, 1), 0)
    any_active = base < nrow

    @pl.when(any_active)
    def _():
        x = w_ref[0] * (ALPHA * s_ref[0])
        m = jnp.max(x, axis=-1, keepdims=True)
        e = jnp.exp(x - m)
        denom = jnp.sum(e, axis=-1, keepdims=True)
        sm = e / denom
        o_ref[0] = jnp.where(row_ids < nrow, sm, 0.0)

    @pl.when(jnp.logical_not(any_active))
    def _():
        o_ref[0] = jnp.zeros_like(o_ref)[0]


def _in_map(b, r, nrow_ref):
    last_active = (nrow_ref[b] - 1) // BR
    return b, jnp.minimum(r, last_active), 0


def _out_map(b, r, nrow_ref):
    return b, r, 0


def kernel(s, nrow_gt, W1):
    B, N, M = s.shape
    nrow = nrow_gt.astype(jnp.int32)
    grid_spec = pltpu.PrefetchScalarGridSpec(
        num_scalar_prefetch=1,
        grid=(B, N // BR),
        in_specs=[
            pl.BlockSpec((1, BR, M), _in_map),
            pl.BlockSpec((1, BR, M), _in_map),
        ],
        out_specs=pl.BlockSpec((1, BR, M), _out_map),
    )
    return pl.pallas_call(
        _body,
        grid_spec=grid_spec,
        out_shape=jax.ShapeDtypeStruct((B, N, M), jnp.float32),
    )(nrow, s, W1)
